# baseline (speedup 1.0000x reference)
import functools

import jax
import jax.numpy as jnp
from jax import lax
from jax.experimental import pallas as pl
from jax.experimental.pallas import tpu as pltpu
from jax.experimental.pallas import tpu_sc as plsc

N = 10000
E = 160000
D = 256
H = 128
OUT = 2

NC = 2
NS = 16
NW = NC * NS
CHUNK = 128
NBUF = 2
EPAD = 163840
ECH = EPAD // CHUNK
CPT = ECH // NW
MAIN_T = (CPT - NBUF) // NBUF
NPAD = 10240
RPT = NPAD // NS

_mesh = lambda: plsc.VectorSubcoreMesh(core_axis_name="c", subcore_axis_name="s")


DW = 128


def _make_deg_kernel():
    @functools.partial(
        pl.kernel,
        out_type=jax.ShapeDtypeStruct((NC, NPAD, DW), jnp.float32),
        mesh=_mesh(),
        scratch_types=[
            pltpu.VMEM((CPT, CHUNK), jnp.int32),
            pltpu.VMEM((CHUNK, DW), jnp.float32),
            pltpu.VMEM_SHARED((NPAD, DW), jnp.float32),
        ],
    )
    def deg_kernel(dst2d_hbm, ones_hbm, z1d_hbm, out_hbm, didx, ones, acc):
        c = lax.axis_index("c")
        s = lax.axis_index("s")
        wid = s * NC + c
        r0 = pl.multiple_of(s * RPT, 8)
        pltpu.sync_copy(dst2d_hbm.at[pl.ds(wid * CPT, CPT)], didx)
        pltpu.sync_copy(ones_hbm, ones)
        pltpu.sync_copy(z1d_hbm, acc.at[pl.ds(r0, RPT)])
        plsc.subcore_barrier()

        def body(i, _):
            pltpu.sync_copy(ones, acc.at[didx.at[i]], add=True)
            return 0

        lax.fori_loop(0, CPT, body, 0)
        plsc.subcore_barrier()
        pltpu.sync_copy(acc.at[pl.ds(r0, RPT)], out_hbm.at[c, pl.ds(r0, RPT)])

    return deg_kernel


def _make_agg_kernel(F):
    @functools.partial(
        pl.kernel,
        out_type=jax.ShapeDtypeStruct((NC, NPAD, F), jnp.float32),
        mesh=_mesh(),
        scratch_types=[
            pltpu.VMEM((CPT, CHUNK), jnp.int32),
            pltpu.VMEM((CPT, CHUNK), jnp.int32),
            pltpu.VMEM((CHUNK, F), jnp.float32),
            pltpu.VMEM_SHARED((NPAD, F), jnp.float32),
            pltpu.SemaphoreType.DMA,
        ],
    )
    def agg_kernel(tbl_hbm, src2d_hbm, dst2d_hbm, z2d_hbm, out_hbm,
                   sidx, didx, rows, acc, gsem):
        c = lax.axis_index("c")
        s = lax.axis_index("s")
        wid = c * NS + s
        r0 = pl.multiple_of(s * RPT, 8)
        pltpu.sync_copy(src2d_hbm.at[pl.ds(wid * CPT, CPT)], sidx)
        pltpu.sync_copy(dst2d_hbm.at[pl.ds(wid * CPT, CPT)], didx)
        pltpu.sync_copy(z2d_hbm, acc.at[pl.ds(r0, RPT)])
        plsc.subcore_barrier()

        def body(j, _):
            pltpu.async_copy(tbl_hbm.at[sidx.at[j]], rows, gsem).wait()
            pltpu.sync_copy(rows, acc.at[didx.at[j]], add=True)
            return 0

        lax.fori_loop(0, CPT, body, 0)
        plsc.subcore_barrier()
        pltpu.sync_copy(acc.at[pl.ds(r0, RPT)], out_hbm.at[c, pl.ds(r0, RPT)])

    return agg_kernel


_BR = 1000


def _dinv_from_deg(deg_ref):
    d = deg_ref[0, :, 0:1] + deg_ref[1, :, 0:1] + 1.0
    y = lax.rsqrt(d)
    return y * (1.5 - 0.5 * d * y * y)


def _tc1_body(x_ref, w1_ref, deg_ref, hs1_ref):
    dinv = _dinv_from_deg(deg_ref)
    h = jnp.dot(x_ref[...], w1_ref[...], preferred_element_type=jnp.float32,
                 precision=lax.Precision.HIGHEST)
    hs1_ref[...] = h * dinv


def _tc2_body(acc_ref, hs1_ref, deg_ref, b1_ref, g_ref):
    dinv = _dinv_from_deg(deg_ref)
    pre = (acc_ref[0] + acc_ref[1] + hs1_ref[...]) * dinv + b1_ref[...]
    z = jnp.maximum(pre, 0.0)
    g_ref[...] = z * dinv


def _tc3_body(acc_ref, g_ref, deg_ref, w2_ref, b2_ref, out_ref):
    dinv = _dinv_from_deg(deg_ref)
    tot = acc_ref[0] + acc_ref[1] + g_ref[...]
    h2 = jnp.dot(tot, w2_ref[...], preferred_element_type=jnp.float32,
                 precision=lax.Precision.HIGHEST)
    out_ref[...] = h2 * dinv + b2_ref[...]


def kernel(x, edge_index, W1, b1, W2, b2):
    x = x.astype(jnp.float32)
    npad_e = EPAD - E
    sent_dst = N + (jnp.arange(npad_e, dtype=jnp.int32) % (NPAD - N))
    src2d = jnp.concatenate(
        [edge_index[0], jnp.zeros((npad_e,), jnp.int32)]).reshape(ECH, CHUNK)
    dst2d = jnp.concatenate([edge_index[1], sent_dst]).reshape(ECH, CHUNK)
    ones2d = jnp.ones((CHUNK, DW), jnp.float32)
    z2d_h = jnp.zeros((RPT, H), jnp.float32)
    z1d = z2d_h

    deg_p = _make_deg_kernel()(dst2d, ones2d, z1d)
    _deg_spec = pl.BlockSpec((NC, _BR, DW), lambda i: (0, i, 0))
    _acc_spec = pl.BlockSpec((NC, _BR, H), lambda i: (0, i, 0))

    hs1 = pl.pallas_call(
        _tc1_body,
        grid=(N // _BR,),
        in_specs=[
            pl.BlockSpec((_BR, D), lambda i: (i, 0)),
            pl.BlockSpec((D, H), lambda i: (0, 0)),
            _deg_spec,
        ],
        out_specs=pl.BlockSpec((_BR, H), lambda i: (i, 0)),
        out_shape=jax.ShapeDtypeStruct((N, H), jnp.float32),
    )(x, W1, deg_p)

    acc1 = _make_agg_kernel(H)(hs1, src2d, dst2d, z2d_h)

    g = pl.pallas_call(
        _tc2_body,
        grid=(N // _BR,),
        in_specs=[
            _acc_spec,
            pl.BlockSpec((_BR, H), lambda i: (i, 0)),
            _deg_spec,
            pl.BlockSpec((1, H), lambda i: (0, 0)),
        ],
        out_specs=pl.BlockSpec((_BR, H), lambda i: (i, 0)),
        out_shape=jax.ShapeDtypeStruct((N, H), jnp.float32),
    )(acc1, hs1, deg_p, b1.reshape(1, H))

    acc2 = _make_agg_kernel(H)(g, src2d, dst2d, z2d_h)

    out = pl.pallas_call(
        _tc3_body,
        grid=(N // _BR,),
        in_specs=[
            _acc_spec,
            pl.BlockSpec((_BR, H), lambda i: (i, 0)),
            _deg_spec,
            pl.BlockSpec((H, OUT), lambda i: (0, 0)),
            pl.BlockSpec((1, OUT), lambda i: (0, 0)),
        ],
        out_specs=pl.BlockSpec((_BR, OUT), lambda i: (i, 0)),
        out_shape=jax.ShapeDtypeStruct((N, OUT), jnp.float32),
    )(acc2, g, deg_p, W2, b2.reshape(1, OUT))

    return out


# speedup vs baseline: 2.1975x; 2.1975x over previous
import functools

import jax
import jax.numpy as jnp
from jax import lax
from jax.experimental import pallas as pl
from jax.experimental.pallas import tpu as pltpu
from jax.experimental.pallas import tpu_sc as plsc

N = 10000
E = 160000
D = 256
H = 128
OUT = 2

NC = 2
NS = 16
NW = NC * NS
CHUNK = 128
NBUF = 2
EPAD = 163840
ECH = EPAD // CHUNK
CPT = ECH // NW
MAIN_T = (CPT - NBUF) // NBUF
NPAD = 10240
RPT = NPAD // NS

_mesh = lambda: plsc.VectorSubcoreMesh(core_axis_name="c", subcore_axis_name="s")


DW = 128


def _make_deg_kernel():
    @functools.partial(
        pl.kernel,
        out_type=jax.ShapeDtypeStruct((NC, NPAD, DW), jnp.float32),
        mesh=_mesh(),
        scratch_types=[
            pltpu.VMEM((CPT, CHUNK), jnp.int32),
            pltpu.VMEM((CHUNK, DW), jnp.float32),
            pltpu.VMEM_SHARED((NPAD, DW), jnp.float32),
        ],
    )
    def deg_kernel(dst2d_hbm, ones_hbm, z1d_hbm, out_hbm, didx, ones, acc):
        c = lax.axis_index("c")
        s = lax.axis_index("s")
        wid = s * NC + c
        r0 = pl.multiple_of(s * RPT, 8)
        pltpu.sync_copy(dst2d_hbm.at[pl.ds(wid * CPT, CPT)], didx)
        pltpu.sync_copy(ones_hbm, ones)
        pltpu.sync_copy(z1d_hbm, acc.at[pl.ds(r0, RPT)])
        plsc.subcore_barrier()

        def body(i, _):
            pltpu.sync_copy(ones, acc.at[didx.at[i]], add=True)
            return 0

        lax.fori_loop(0, CPT, body, 0)
        plsc.subcore_barrier()
        pltpu.sync_copy(acc.at[pl.ds(r0, RPT)], out_hbm.at[c, pl.ds(r0, RPT)])

    return deg_kernel


def _make_agg_kernel(F):
    @functools.partial(
        pl.kernel,
        out_type=jax.ShapeDtypeStruct((NC, NPAD, F), jnp.float32),
        mesh=_mesh(),
        scratch_types=[
            pltpu.VMEM((CPT, CHUNK), jnp.int32),
            pltpu.VMEM((CPT, CHUNK), jnp.int32),
            pltpu.VMEM((CHUNK, F), jnp.float32),
            pltpu.VMEM_SHARED((NPAD, F), jnp.float32),
            pltpu.SemaphoreType.DMA,
        ],
    )
    def agg_kernel(tbl_hbm, src2d_hbm, dst2d_hbm, z2d_hbm, out_hbm,
                   sidx, didx, rows, acc, gsem):
        c = lax.axis_index("c")
        s = lax.axis_index("s")
        wid = c * NS + s
        r0 = pl.multiple_of(s * RPT, 8)
        pltpu.sync_copy(src2d_hbm.at[pl.ds(wid * CPT, CPT)], sidx)
        pltpu.sync_copy(dst2d_hbm.at[pl.ds(wid * CPT, CPT)], didx)
        pltpu.sync_copy(z2d_hbm, acc.at[pl.ds(r0, RPT)])
        plsc.subcore_barrier()

        def body(j, _):
            pltpu.async_copy(tbl_hbm.at[sidx.at[j]], rows, gsem).wait()
            pltpu.sync_copy(rows, acc.at[didx.at[j]], add=True)
            return 0

        lax.fori_loop(0, CPT, body, 0)
        plsc.subcore_barrier()
        pltpu.sync_copy(acc.at[pl.ds(r0, RPT)], out_hbm.at[c, pl.ds(r0, RPT)])

    return agg_kernel


_BR = 1000


def _dinv_from_deg(deg_ref):
    d = deg_ref[0, :, 0:1] + deg_ref[1, :, 0:1] + 1.0
    y = lax.rsqrt(d)
    return y * (1.5 - 0.5 * d * y * y)


def _tc1_body(x_ref, w1_ref, deg_ref, hs1_ref):
    dinv = _dinv_from_deg(deg_ref)
    h = jnp.dot(x_ref[...], w1_ref[...], preferred_element_type=jnp.float32,
                 precision=lax.Precision.HIGHEST)
    hs1_ref[...] = h * dinv


def _tc2_body(acc_ref, hs1_ref, deg_ref, b1_ref, g_ref):
    dinv = _dinv_from_deg(deg_ref)
    pre = (acc_ref[0] + acc_ref[1] + hs1_ref[...]) * dinv + b1_ref[...]
    z = jnp.maximum(pre, 0.0)
    g_ref[...] = z * dinv


def _tc3_body(acc_ref, g_ref, deg_ref, w2_ref, b2_ref, out_ref):
    dinv = _dinv_from_deg(deg_ref)
    tot = acc_ref[0] + acc_ref[1] + g_ref[...]
    h2 = jnp.dot(tot, w2_ref[...], preferred_element_type=jnp.float32,
                 precision=lax.Precision.HIGHEST)
    out_ref[...] = h2 * dinv + b2_ref[...]


def kernel(x, edge_index, W1, b1, W2, b2):
    x = x.astype(jnp.float32)
    npad_e = EPAD - E
    pad_iota = jnp.arange(npad_e, dtype=jnp.int32)
    sent_src = (pad_iota * 37) % N
    sent_dst = N + pad_iota % (NPAD - N)
    src2d = jnp.concatenate([edge_index[0], sent_src]).reshape(ECH, CHUNK)
    dst2d = jnp.concatenate([edge_index[1], sent_dst]).reshape(ECH, CHUNK)
    ones2d = jnp.ones((CHUNK, DW), jnp.float32)
    z2d_h = jnp.zeros((RPT, H), jnp.float32)
    z1d = z2d_h

    deg_p = _make_deg_kernel()(dst2d, ones2d, z1d)
    _deg_spec = pl.BlockSpec((NC, _BR, DW), lambda i: (0, i, 0))
    _acc_spec = pl.BlockSpec((NC, _BR, H), lambda i: (0, i, 0))

    hs1 = pl.pallas_call(
        _tc1_body,
        grid=(N // _BR,),
        in_specs=[
            pl.BlockSpec((_BR, D), lambda i: (i, 0)),
            pl.BlockSpec((D, H), lambda i: (0, 0)),
            _deg_spec,
        ],
        out_specs=pl.BlockSpec((_BR, H), lambda i: (i, 0)),
        out_shape=jax.ShapeDtypeStruct((N, H), jnp.float32),
    )(x, W1, deg_p)

    acc1 = _make_agg_kernel(H)(hs1, src2d, dst2d, z2d_h)

    g = pl.pallas_call(
        _tc2_body,
        grid=(N // _BR,),
        in_specs=[
            _acc_spec,
            pl.BlockSpec((_BR, H), lambda i: (i, 0)),
            _deg_spec,
            pl.BlockSpec((1, H), lambda i: (0, 0)),
        ],
        out_specs=pl.BlockSpec((_BR, H), lambda i: (i, 0)),
        out_shape=jax.ShapeDtypeStruct((N, H), jnp.float32),
    )(acc1, hs1, deg_p, b1.reshape(1, H))

    acc2 = _make_agg_kernel(H)(g, src2d, dst2d, z2d_h)

    out = pl.pallas_call(
        _tc3_body,
        grid=(N // _BR,),
        in_specs=[
            _acc_spec,
            pl.BlockSpec((_BR, H), lambda i: (i, 0)),
            _deg_spec,
            pl.BlockSpec((H, OUT), lambda i: (0, 0)),
            pl.BlockSpec((1, OUT), lambda i: (0, 0)),
        ],
        out_specs=pl.BlockSpec((_BR, OUT), lambda i: (i, 0)),
        out_shape=jax.ShapeDtypeStruct((N, OUT), jnp.float32),
    )(acc2, g, deg_p, W2, b2.reshape(1, OUT))

    return out


# speedup vs baseline: 2.8122x; 1.2797x over previous
import functools

import jax
import jax.numpy as jnp
from jax import lax
from jax.experimental import pallas as pl
from jax.experimental.pallas import tpu as pltpu
from jax.experimental.pallas import tpu_sc as plsc

N = 10000
E = 160000
D = 256
H = 128
OUT = 2

NC = 2
NS = 16
NW = NC * NS
CHUNK = 128
NBUF = 2
EPAD = 163840
ECH = EPAD // CHUNK
CPT = ECH // NW
MAIN_T = (CPT - NBUF) // NBUF
NPAD = 10240
RPT = NPAD // NS

_mesh = lambda: plsc.VectorSubcoreMesh(core_axis_name="c", subcore_axis_name="s")


DW = 128


def _make_deg_kernel():
    @functools.partial(
        pl.kernel,
        out_type=jax.ShapeDtypeStruct((NC, NPAD, DW), jnp.float32),
        mesh=_mesh(),
        scratch_types=[
            pltpu.VMEM((CPT, CHUNK), jnp.int32),
            pltpu.VMEM((CHUNK, DW), jnp.float32),
            pltpu.VMEM_SHARED((NPAD, DW), jnp.float32),
        ],
    )
    def deg_kernel(dst2d_hbm, ones_hbm, z1d_hbm, out_hbm, didx, ones, acc):
        c = lax.axis_index("c")
        s = lax.axis_index("s")
        wid = s * NC + c
        r0 = pl.multiple_of(s * RPT, 8)
        pltpu.sync_copy(dst2d_hbm.at[pl.ds(wid * CPT, CPT)], didx)
        pltpu.sync_copy(ones_hbm, ones)
        pltpu.sync_copy(z1d_hbm, acc.at[pl.ds(r0, RPT)])
        plsc.subcore_barrier()

        def body(i, _):
            pltpu.sync_copy(ones, acc.at[didx.at[i]], add=True)
            return 0

        lax.fori_loop(0, CPT, body, 0)
        plsc.subcore_barrier()
        pltpu.sync_copy(acc.at[pl.ds(r0, RPT)], out_hbm.at[c, pl.ds(r0, RPT)])

    return deg_kernel


def _make_agg_kernel(F):
    @functools.partial(
        pl.kernel,
        out_type=jax.ShapeDtypeStruct((NC, NPAD, F), jnp.float32),
        mesh=_mesh(),
        scratch_types=[
            pltpu.VMEM((CPT, CHUNK), jnp.int32),
            pltpu.VMEM((CPT, CHUNK), jnp.int32),
            pltpu.VMEM((2, CHUNK, F), jnp.float32),
            pltpu.VMEM_SHARED((NPAD, F), jnp.float32),
        ] + [pltpu.SemaphoreType.DMA] * 2,
    )
    def agg_kernel(tbl_hbm, src2d_hbm, dst2d_hbm, z2d_hbm, out_hbm,
                   sidx, didx, rows, acc, *gsems):
        c = lax.axis_index("c")
        s = lax.axis_index("s")
        wid = c * NS + s
        r0 = pl.multiple_of(s * RPT, 8)
        pltpu.sync_copy(src2d_hbm.at[pl.ds(wid * CPT, CPT)], sidx)
        pltpu.sync_copy(dst2d_hbm.at[pl.ds(wid * CPT, CPT)], didx)

        def gather_start(j, q):
            pltpu.async_copy(tbl_hbm.at[sidx.at[j]], rows.at[q], gsems[q])

        def gather_wait(q):
            pltpu.make_async_copy(tbl_hbm.at[pl.ds(0, CHUNK)], rows.at[q],
                                  gsems[q]).wait()

        def scatter(j, q):
            gather_wait(q)
            pltpu.sync_copy(rows.at[q], acc.at[didx.at[j]], add=True)

        gather_start(0, 0)
        gather_start(1, 1)
        pltpu.sync_copy(z2d_hbm, acc.at[pl.ds(r0, RPT)])
        plsc.subcore_barrier()

        def body(t, _):
            j0 = t * 2
            for u in range(2):
                scatter(j0 + u, u)
                gather_start(j0 + u + 2, u)
            return 0

        lax.fori_loop(0, CPT // 2 - 1, body, 0)
        scatter(CPT - 2, 0)
        scatter(CPT - 1, 1)
        plsc.subcore_barrier()
        pltpu.sync_copy(acc.at[pl.ds(r0, RPT)], out_hbm.at[c, pl.ds(r0, RPT)])

    return agg_kernel


_BR = 1000


def _dinv_from_deg(deg_ref):
    d = deg_ref[0, :, 0:1] + deg_ref[1, :, 0:1] + 1.0
    y = lax.rsqrt(d)
    return y * (1.5 - 0.5 * d * y * y)


def _tc1_body(x_ref, w1_ref, deg_ref, hs1_ref):
    dinv = _dinv_from_deg(deg_ref)
    h = jnp.dot(x_ref[...], w1_ref[...], preferred_element_type=jnp.float32,
                 precision=lax.Precision.HIGHEST)
    hs1_ref[...] = h * dinv


def _tc2_body(acc_ref, hs1_ref, deg_ref, b1_ref, g_ref):
    dinv = _dinv_from_deg(deg_ref)
    pre = (acc_ref[0] + acc_ref[1] + hs1_ref[...]) * dinv + b1_ref[...]
    z = jnp.maximum(pre, 0.0)
    g_ref[...] = z * dinv


def _tc3_body(acc_ref, g_ref, deg_ref, w2_ref, b2_ref, out_ref):
    dinv = _dinv_from_deg(deg_ref)
    tot = acc_ref[0] + acc_ref[1] + g_ref[...]
    h2 = jnp.dot(tot, w2_ref[...], preferred_element_type=jnp.float32,
                 precision=lax.Precision.HIGHEST)
    out_ref[...] = h2 * dinv + b2_ref[...]


def kernel(x, edge_index, W1, b1, W2, b2):
    x = x.astype(jnp.float32)
    npad_e = EPAD - E
    pad_iota = jnp.arange(npad_e, dtype=jnp.int32)
    sent_src = (pad_iota * 37) % N
    sent_dst = N + pad_iota % (NPAD - N)
    src2d = jnp.concatenate([edge_index[0], sent_src]).reshape(ECH, CHUNK)
    dst2d = jnp.concatenate([edge_index[1], sent_dst]).reshape(ECH, CHUNK)
    ones2d = jnp.ones((CHUNK, DW), jnp.float32)
    z2d_h = jnp.zeros((RPT, H), jnp.float32)
    z1d = z2d_h

    deg_p = _make_deg_kernel()(dst2d, ones2d, z1d)
    _deg_spec = pl.BlockSpec((NC, _BR, DW), lambda i: (0, i, 0))
    _acc_spec = pl.BlockSpec((NC, _BR, H), lambda i: (0, i, 0))

    hs1 = pl.pallas_call(
        _tc1_body,
        grid=(N // _BR,),
        in_specs=[
            pl.BlockSpec((_BR, D), lambda i: (i, 0)),
            pl.BlockSpec((D, H), lambda i: (0, 0)),
            _deg_spec,
        ],
        out_specs=pl.BlockSpec((_BR, H), lambda i: (i, 0)),
        out_shape=jax.ShapeDtypeStruct((N, H), jnp.float32),
    )(x, W1, deg_p)

    acc1 = _make_agg_kernel(H)(hs1, src2d, dst2d, z2d_h)

    g = pl.pallas_call(
        _tc2_body,
        grid=(N // _BR,),
        in_specs=[
            _acc_spec,
            pl.BlockSpec((_BR, H), lambda i: (i, 0)),
            _deg_spec,
            pl.BlockSpec((1, H), lambda i: (0, 0)),
        ],
        out_specs=pl.BlockSpec((_BR, H), lambda i: (i, 0)),
        out_shape=jax.ShapeDtypeStruct((N, H), jnp.float32),
    )(acc1, hs1, deg_p, b1.reshape(1, H))

    acc2 = _make_agg_kernel(H)(g, src2d, dst2d, z2d_h)

    out = pl.pallas_call(
        _tc3_body,
        grid=(N // _BR,),
        in_specs=[
            _acc_spec,
            pl.BlockSpec((_BR, H), lambda i: (i, 0)),
            _deg_spec,
            pl.BlockSpec((H, OUT), lambda i: (0, 0)),
            pl.BlockSpec((1, OUT), lambda i: (0, 0)),
        ],
        out_specs=pl.BlockSpec((_BR, OUT), lambda i: (i, 0)),
        out_shape=jax.ShapeDtypeStruct((N, OUT), jnp.float32),
    )(acc2, g, deg_p, W2, b2.reshape(1, OUT))

    return out


# speedup vs baseline: 2.8481x; 1.0128x over previous
import functools

import jax
import jax.numpy as jnp
from jax import lax
from jax.experimental import pallas as pl
from jax.experimental.pallas import tpu as pltpu
from jax.experimental.pallas import tpu_sc as plsc

N = 10000
E = 160000
D = 256
H = 128
OUT = 2

NC = 2
NS = 16
NW = NC * NS
CHUNK = 128
NBUF = 2
EPAD = 163840
ECH = EPAD // CHUNK
CPT = ECH // NW
MAIN_T = (CPT - NBUF) // NBUF
NPAD = 10240
RPT = NPAD // NS

_mesh = lambda: plsc.VectorSubcoreMesh(core_axis_name="c", subcore_axis_name="s")


DW = 128


def _make_deg_kernel():
    @functools.partial(
        pl.kernel,
        out_type=jax.ShapeDtypeStruct((NC, NPAD, DW), jnp.float32),
        mesh=_mesh(),
        scratch_types=[
            pltpu.VMEM((CPT, CHUNK), jnp.int32),
            pltpu.VMEM((CHUNK, DW), jnp.float32),
            pltpu.VMEM_SHARED((NPAD, DW), jnp.float32),
        ],
    )
    def deg_kernel(dst2d_hbm, ones_hbm, z1d_hbm, out_hbm, didx, ones, acc):
        c = lax.axis_index("c")
        s = lax.axis_index("s")
        wid = s * NC + c
        r0 = pl.multiple_of(s * RPT, 8)
        pltpu.sync_copy(dst2d_hbm.at[pl.ds(wid * CPT, CPT)], didx)
        pltpu.sync_copy(ones_hbm, ones)
        pltpu.sync_copy(z1d_hbm, acc.at[pl.ds(r0, RPT)])
        plsc.subcore_barrier()

        def body(i, _):
            pltpu.sync_copy(ones, acc.at[didx.at[i]], add=True)
            return 0

        lax.fori_loop(0, CPT, body, 0)
        plsc.subcore_barrier()
        pltpu.sync_copy(acc.at[pl.ds(r0, RPT)], out_hbm.at[c, pl.ds(r0, RPT)])

    return deg_kernel


def _make_agg_kernel(F):
    @functools.partial(
        pl.kernel,
        out_type=jax.ShapeDtypeStruct((NC, NPAD, F), jnp.float32),
        mesh=_mesh(),
        scratch_types=[
            pltpu.VMEM((CPT, CHUNK), jnp.int32),
            pltpu.VMEM((CPT, CHUNK), jnp.int32),
            pltpu.VMEM((2, CHUNK, F), jnp.float32),
            pltpu.VMEM_SHARED((NPAD, F), jnp.float32),
        ] + [pltpu.SemaphoreType.DMA] * 2,
    )
    def agg_kernel(tbl_hbm, src2d_hbm, dst2d_hbm, z2d_hbm, out_hbm,
                   sidx, didx, rows, acc, *gsems):
        c = lax.axis_index("c")
        s = lax.axis_index("s")
        wid = c * NS + s
        r0 = pl.multiple_of(s * RPT, 8)
        pltpu.sync_copy(src2d_hbm.at[pl.ds(wid * CPT, CPT)], sidx)
        pltpu.sync_copy(dst2d_hbm.at[pl.ds(wid * CPT, CPT)], didx)

        def gather_start(j, q):
            pltpu.async_copy(tbl_hbm.at[sidx.at[j]], rows.at[q], gsems[q])

        def gather_wait(q):
            pltpu.make_async_copy(tbl_hbm.at[pl.ds(0, CHUNK)], rows.at[q],
                                  gsems[q]).wait()

        def scatter(j, q):
            gather_wait(q)
            pltpu.sync_copy(rows.at[q], acc.at[didx.at[j]], add=True)

        gather_start(0, 0)
        gather_start(1, 1)
        pltpu.sync_copy(z2d_hbm, acc.at[pl.ds(r0, RPT)])
        plsc.subcore_barrier()

        def body(t, _):
            j0 = t * 2
            for u in range(2):
                scatter(j0 + u, u)
                gather_start(j0 + u + 2, u)
            return 0

        lax.fori_loop(0, CPT // 2 - 1, body, 0)
        scatter(CPT - 2, 0)
        scatter(CPT - 1, 1)
        plsc.subcore_barrier()
        pltpu.sync_copy(acc.at[pl.ds(r0, RPT)], out_hbm.at[c, pl.ds(r0, RPT)])

    return agg_kernel


_BR = 1000


def _dinv_from_deg(deg_ref):
    d = deg_ref[0, :, 0:1] + deg_ref[1, :, 0:1] + 1.0
    y = lax.rsqrt(d)
    return y * (1.5 - 0.5 * d * y * y)


def _tc1a_body(x_ref, w1_ref, h1_ref):
    h1_ref[...] = jnp.dot(x_ref[...], w1_ref[...],
                          preferred_element_type=jnp.float32,
                          precision=lax.Precision.HIGHEST)


def _tc1b_body(h1_ref, deg_ref, hs1_ref):
    dinv = _dinv_from_deg(deg_ref)
    hs1_ref[...] = h1_ref[...] * dinv


def _tc2_body(acc_ref, hs1_ref, deg_ref, b1_ref, g_ref):
    dinv = _dinv_from_deg(deg_ref)
    pre = (acc_ref[0] + acc_ref[1] + hs1_ref[...]) * dinv + b1_ref[...]
    z = jnp.maximum(pre, 0.0)
    g_ref[...] = z * dinv


def _tc3_body(acc_ref, g_ref, deg_ref, w2_ref, b2_ref, out_ref):
    dinv = _dinv_from_deg(deg_ref)
    tot = acc_ref[0] + acc_ref[1] + g_ref[...]
    h2 = jnp.dot(tot, w2_ref[...], preferred_element_type=jnp.float32,
                 precision=lax.Precision.HIGHEST)
    out_ref[...] = h2 * dinv + b2_ref[...]


def kernel(x, edge_index, W1, b1, W2, b2):
    x = x.astype(jnp.float32)
    npad_e = EPAD - E
    pad_iota = jnp.arange(npad_e, dtype=jnp.int32)
    sent_src = (pad_iota * 37) % N
    sent_dst = N + pad_iota % (NPAD - N)
    src2d = jnp.concatenate([edge_index[0], sent_src]).reshape(ECH, CHUNK)
    dst2d = jnp.concatenate([edge_index[1], sent_dst]).reshape(ECH, CHUNK)
    ones2d = jnp.ones((CHUNK, DW), jnp.float32)
    z2d_h = jnp.zeros((RPT, H), jnp.float32)
    z1d = z2d_h

    deg_p = _make_deg_kernel()(dst2d, ones2d, z1d)
    _deg_spec = pl.BlockSpec((NC, _BR, DW), lambda i: (0, i, 0))
    _acc_spec = pl.BlockSpec((NC, _BR, H), lambda i: (0, i, 0))

    h1 = pl.pallas_call(
        _tc1a_body,
        grid=(N // _BR,),
        in_specs=[
            pl.BlockSpec((_BR, D), lambda i: (i, 0)),
            pl.BlockSpec((D, H), lambda i: (0, 0)),
        ],
        out_specs=pl.BlockSpec((_BR, H), lambda i: (i, 0)),
        out_shape=jax.ShapeDtypeStruct((N, H), jnp.float32),
    )(x, W1)

    hs1 = pl.pallas_call(
        _tc1b_body,
        grid=(N // _BR,),
        in_specs=[
            pl.BlockSpec((_BR, H), lambda i: (i, 0)),
            _deg_spec,
        ],
        out_specs=pl.BlockSpec((_BR, H), lambda i: (i, 0)),
        out_shape=jax.ShapeDtypeStruct((N, H), jnp.float32),
    )(h1, deg_p)

    acc1 = _make_agg_kernel(H)(hs1, src2d, dst2d, z2d_h)

    g = pl.pallas_call(
        _tc2_body,
        grid=(N // _BR,),
        in_specs=[
            _acc_spec,
            pl.BlockSpec((_BR, H), lambda i: (i, 0)),
            _deg_spec,
            pl.BlockSpec((1, H), lambda i: (0, 0)),
        ],
        out_specs=pl.BlockSpec((_BR, H), lambda i: (i, 0)),
        out_shape=jax.ShapeDtypeStruct((N, H), jnp.float32),
    )(acc1, hs1, deg_p, b1.reshape(1, H))

    acc2 = _make_agg_kernel(H)(g, src2d, dst2d, z2d_h)

    out = pl.pallas_call(
        _tc3_body,
        grid=(N // _BR,),
        in_specs=[
            _acc_spec,
            pl.BlockSpec((_BR, H), lambda i: (i, 0)),
            _deg_spec,
            pl.BlockSpec((H, OUT), lambda i: (0, 0)),
            pl.BlockSpec((1, OUT), lambda i: (0, 0)),
        ],
        out_specs=pl.BlockSpec((_BR, OUT), lambda i: (i, 0)),
        out_shape=jax.ShapeDtypeStruct((N, OUT), jnp.float32),
    )(acc2, g, deg_p, W2, b2.reshape(1, OUT))

    return out
